# baseline (speedup 1.0000x reference)
import functools

import jax
import jax.numpy as jnp
from jax import lax
from jax.experimental import pallas as pl
from jax.experimental.pallas import tpu as pltpu
from jax.experimental.pallas import tpu_sc as plsc


def kernel(x, table):
    B, L, D = x.shape
    NC, NS = 2, 16
    NW = NC * NS
    POS_PER_TILE = L // NW
    CH = 8
    NCHUNK = POS_PER_TILE // CH
    W = CH * D

    mesh = plsc.VectorSubcoreMesh(core_axis_name="c", subcore_axis_name="s")

    @functools.partial(
        pl.kernel,
        mesh=mesh,
        out_type=jax.ShapeDtypeStruct((B, L, D), jnp.float32),
        scratch_types=[
            pltpu.VMEM((3, CH, D), jnp.float32),
            pltpu.VMEM((3, B, CH, D), jnp.float32),
            pltpu.SemaphoreType.DMA,
            pltpu.SemaphoreType.DMA,
            pltpu.SemaphoreType.DMA,
            pltpu.SemaphoreType.DMA,
            pltpu.SemaphoreType.DMA,
            pltpu.SemaphoreType.DMA,
        ],
        compiler_params=pltpu.CompilerParams(use_tc_tiling_on_sc=True),
    )
    def sc_add(x_hbm, t_hbm, o_hbm, tbuf, xbuf, in0, in1, in2, out0, out1, out2):
        wid = lax.axis_index("s") * NC + lax.axis_index("c")
        base = wid * POS_PER_TILE
        in_sem = [in0, in1, in2]
        out_sem = [out0, out1, out2]

        def start_in(c, s):
            pos = base + c * CH
            return [
                pltpu.async_copy(t_hbm.at[pl.ds(pos, CH), :], tbuf.at[s], in_sem[s]),
                pltpu.async_copy(
                    x_hbm.at[:, pl.ds(pos, CH), :], xbuf.at[s], in_sem[s]
                ),
            ]

        def start_out(c, s):
            pos = base + c * CH
            return [
                pltpu.async_copy(
                    xbuf.at[s], o_hbm.at[:, pl.ds(pos, CH), :], out_sem[s]
                )
            ]

        def compute(s):
            @plsc.parallel_loop(0, W, 16, unroll=16)
            def _(i):
                r = i >> 10
                o = pl.multiple_of(i & (D - 1), 16)
                t = tbuf[s, r, pl.ds(o, 16)]
                for b in range(B):
                    xbuf[s, b, r, pl.ds(o, 16)] = xbuf[s, b, r, pl.ds(o, 16)] + t

        pend_in = [None, None, None]
        pend_out = [None, None, None]
        pend_in[0] = start_in(0, 0)
        pend_in[1] = start_in(1, 1)
        for c in range(NCHUNK):
            s = c % 3
            nc = c + 2
            if nc < NCHUNK:
                ns = nc % 3
                if pend_out[ns] is not None:
                    for h in pend_out[ns]:
                        h.wait()
                    pend_out[ns] = None
                pend_in[ns] = start_in(nc, ns)
            for h in pend_in[s]:
                h.wait()
            compute(s)
            pend_out[s] = start_out(c, s)
        for s in range(3):
            if pend_out[s] is not None:
                for h in pend_out[s]:
                    h.wait()

    return sc_add(x, table)


# speedup vs baseline: 1.0263x; 1.0263x over previous
import functools

import jax
import jax.numpy as jnp
from jax import lax
from jax.experimental import pallas as pl
from jax.experimental.pallas import tpu as pltpu
from jax.experimental.pallas import tpu_sc as plsc


def kernel(x, table):
    B, L, D = x.shape
    NC, NS = 2, 16
    NW = NC * NS
    POS_PER_TILE = L // NW
    CH = 8
    NCHUNK = POS_PER_TILE // CH
    W = CH * D

    mesh = plsc.VectorSubcoreMesh(core_axis_name="c", subcore_axis_name="s")

    @functools.partial(
        pl.kernel,
        mesh=mesh,
        out_type=jax.ShapeDtypeStruct((B, L, D), jnp.float32),
        scratch_types=[
            pltpu.VMEM((3, CH, D), jnp.float32),
            pltpu.VMEM((3, B, CH, D), jnp.float32),
            pltpu.SemaphoreType.DMA,
            pltpu.SemaphoreType.DMA,
            pltpu.SemaphoreType.DMA,
            pltpu.SemaphoreType.DMA,
            pltpu.SemaphoreType.DMA,
            pltpu.SemaphoreType.DMA,
        ],
        compiler_params=pltpu.CompilerParams(use_tc_tiling_on_sc=True),
    )
    def sc_add(x_hbm, t_hbm, o_hbm, tbuf, xbuf, in0, in1, in2, out0, out1, out2):
        wid = lax.axis_index("c") * NS + lax.axis_index("s")
        base = wid * POS_PER_TILE
        in_sem = [in0, in1, in2]
        out_sem = [out0, out1, out2]

        def start_in(c, s):
            pos = base + c * CH
            return [
                pltpu.async_copy(t_hbm.at[pl.ds(pos, CH), :], tbuf.at[s], in_sem[s]),
                pltpu.async_copy(
                    x_hbm.at[:, pl.ds(pos, CH), :], xbuf.at[s], in_sem[s]
                ),
            ]

        def start_out(c, s):
            pos = base + c * CH
            return [
                pltpu.async_copy(
                    xbuf.at[s], o_hbm.at[:, pl.ds(pos, CH), :], out_sem[s]
                )
            ]

        def compute(s):
            @plsc.parallel_loop(0, W, 16, unroll=8)
            def _(i):
                r = i >> 10
                o = pl.multiple_of(i & (D - 1), 16)
                t = tbuf[s, r, pl.ds(o, 16)]
                for b in range(B):
                    xbuf[s, b, r, pl.ds(o, 16)] = xbuf[s, b, r, pl.ds(o, 16)] + t

        pend_in = [None, None, None]
        pend_out = [None, None, None]
        pend_in[0] = start_in(0, 0)
        pend_in[1] = start_in(1, 1)
        for c in range(NCHUNK):
            s = c % 3
            nc = c + 2
            if nc < NCHUNK:
                ns = nc % 3
                if pend_out[ns] is not None:
                    for h in pend_out[ns]:
                        h.wait()
                    pend_out[ns] = None
                pend_in[ns] = start_in(nc, ns)
            for h in pend_in[s]:
                h.wait()
            compute(s)
            pend_out[s] = start_out(c, s)
        for s in range(3):
            if pend_out[s] is not None:
                for h in pend_out[s]:
                    h.wait()

    return sc_add(x, table)
